# baseline (speedup 1.0000x reference)
import functools

import jax
import jax.numpy as jnp
from jax import lax
from jax.experimental import pallas as pl
from jax.experimental.pallas import tpu as pltpu
from jax.experimental.pallas import tpu_sc as plsc

_NC = 2
_NS = 16
_NW = _NC * _NS
_L = 16

_BATCH = 16384
_FEAT = 64
_B_PER_W = _BATCH // _NW
_CHUNKS = _FEAT // _L


def _sc_partials(features, labels, centers):
    mesh = plsc.VectorSubcoreMesh(
        core_axis_name="c", subcore_axis_name="s",
        num_cores=_NC, num_subcores=_NS,
    )

    @functools.partial(
        pl.kernel,
        out_type=jax.ShapeDtypeStruct((_NW, _L), jnp.float32),
        mesh=mesh,
        scratch_types=[
            pltpu.VMEM((_B_PER_W,), jnp.int32),
            pltpu.VMEM((_B_PER_W, _FEAT), jnp.float32),
            pltpu.VMEM((_B_PER_W, _FEAT), jnp.float32),
            pltpu.VMEM((_L,), jnp.float32),
            pltpu.SemaphoreType.DMA,
            pltpu.SemaphoreType.DMA,
        ],
        compiler_params=pltpu.CompilerParams(use_tc_tiling_on_sc=False),
    )
    def k(feat_hbm, lab_hbm, cent_hbm, out_hbm,
          idx_v, rows_v, feat_v, acc_v, sem_g, sem_f):
        wid = lax.axis_index("s") * _NC + lax.axis_index("c")
        base = wid * _B_PER_W

        pltpu.sync_copy(lab_hbm.at[pl.ds(base, _B_PER_W)], idx_v)
        gather = pltpu.async_copy(cent_hbm.at[idx_v], rows_v, sem_g)
        feats = pltpu.async_copy(
            feat_hbm.at[pl.ds(base, _B_PER_W)], feat_v, sem_f)
        feats.wait()
        gather.wait()

        def body(i, carry):
            out = []
            for c in range(_CHUNKS):
                f = feat_v[i, pl.ds(c * _L, _L)]
                g = rows_v[i, pl.ds(c * _L, _L)]
                d = f - g
                out.append(carry[c] + d * d)
            return tuple(out)

        zero = jnp.zeros((_L,), jnp.float32)
        accs = lax.fori_loop(0, _B_PER_W, body, (zero,) * _CHUNKS)
        total = accs[0] + accs[1] + accs[2] + accs[3]
        acc_v[...] = total
        pltpu.sync_copy(acc_v, out_hbm.at[wid])

    return k(features, labels, centers)


def _reduce_body(p_ref, o_ref):
    o_ref[0, 0] = jnp.sum(p_ref[...]) * (0.5 / _BATCH)


def _final_reduce(partials):
    out = pl.pallas_call(
        _reduce_body,
        out_shape=jax.ShapeDtypeStruct((1, 1), jnp.float32),
        out_specs=pl.BlockSpec(memory_space=pltpu.SMEM),
    )(partials)
    return out[0, 0]


def kernel(features, labels, centers):
    labels = labels.astype(jnp.int32)
    partials = _sc_partials(features, labels, centers)
    return _final_reduce(partials)


# speedup vs baseline: 1.3154x; 1.3154x over previous
import functools

import jax
import jax.numpy as jnp
from jax import lax
from jax.experimental import pallas as pl
from jax.experimental.pallas import tpu as pltpu
from jax.experimental.pallas import tpu_sc as plsc

_NC = 2
_NS = 16
_NW = _NC * _NS
_L = 16

_BATCH = 16384
_FEAT = 64
_B_PER_W = _BATCH // _NW
_COLC = _FEAT // _L
_G = 16
_NG = _B_PER_W // _G
_RING = 4
_LAG = 2


def _sc_partials(features, labels, centers):
    mesh = plsc.VectorSubcoreMesh(
        core_axis_name="c", subcore_axis_name="s",
        num_cores=_NC, num_subcores=_NS,
    )

    @functools.partial(
        pl.kernel,
        out_type=jax.ShapeDtypeStruct((_NW, _L), jnp.float32),
        mesh=mesh,
        scratch_types=[
            pltpu.VMEM((_B_PER_W,), jnp.int32),
            pltpu.VMEM((_RING, _G, _FEAT), jnp.float32),
            pltpu.VMEM((_RING, _G, _FEAT), jnp.float32),
            pltpu.VMEM((_L,), jnp.float32),
            pltpu.SemaphoreType.DMA,
            pltpu.SemaphoreType.DMA,
        ],
        compiler_params=pltpu.CompilerParams(
            use_tc_tiling_on_sc=True, needs_layout_passes=False),
    )
    def k(feat_hbm, lab_hbm, cent_hbm, out_hbm,
          lab_v, rows_v, frows_v, acc_v, sem_g, sem_f):
        wid = lax.axis_index("s") * _NC + lax.axis_index("c")
        base = wid * _B_PER_W

        pltpu.sync_copy(lab_hbm.at[pl.ds(base, _B_PER_W)], lab_v)
        lanes = lax.iota(jnp.int32, _L)

        def fire(g):
            slot = lax.rem(g, _RING)
            labv = lab_v[pl.ds(g * _G, _G)]
            for i in range(_G):
                s = lax.reduce_sum(
                    jnp.where(lanes == i, labv, 0), axes=(0,))
                pltpu.async_copy(cent_hbm.at[s], rows_v.at[slot, i], sem_g)
            pltpu.async_copy(
                feat_hbm.at[pl.ds(base + g * _G, _G)],
                frows_v.at[slot], sem_f)

        def drain_and_accum(g, accs):
            slot = lax.rem(g, _RING)
            for i in range(_G):
                pltpu.make_async_copy(
                    cent_hbm.at[0], rows_v.at[slot, i], sem_g).wait()
            pltpu.make_async_copy(
                feat_hbm.at[pl.ds(0, _G)], frows_v.at[slot], sem_f).wait()

            def row_body(i, a):
                out = []
                for c in range(_COLC):
                    cv = rows_v[slot, i, pl.ds(c * _L, _L)]
                    fv = frows_v[slot, i, pl.ds(c * _L, _L)]
                    d = fv - cv
                    out.append(a[c] + d * d)
                return tuple(out)

            return lax.fori_loop(0, _G, row_body, accs)

        def step(g, accs):
            accs = lax.cond(
                g < _NG, lambda a: (fire(g), a)[1], lambda a: a, accs)
            return lax.cond(
                g >= _LAG, lambda a: drain_and_accum(g - _LAG, a),
                lambda a: a, accs)

        zero = jnp.zeros((_L,), jnp.float32)
        accs = lax.fori_loop(0, _NG + _LAG, step, (zero,) * _COLC)
        acc_v[...] = accs[0] + accs[1] + accs[2] + accs[3]
        pltpu.sync_copy(acc_v, out_hbm.at[wid])

    return k(features, labels, centers)


def _reduce_body(p_ref, o_ref):
    o_ref[0, 0] = jnp.sum(p_ref[...]) * (0.5 / _BATCH)


def _final_reduce(partials):
    out = pl.pallas_call(
        _reduce_body,
        out_shape=jax.ShapeDtypeStruct((1, 1), jnp.float32),
        out_specs=pl.BlockSpec(memory_space=pltpu.SMEM),
    )(partials)
    return out[0, 0]


def kernel(features, labels, centers):
    labels = labels.astype(jnp.int32)
    partials = _sc_partials(features, labels, centers)
    return _final_reduce(partials)


# speedup vs baseline: 2.0756x; 1.5779x over previous
import functools

import jax
import jax.numpy as jnp
from jax import lax
from jax.experimental import pallas as pl
from jax.experimental.pallas import tpu as pltpu
from jax.experimental.pallas import tpu_sc as plsc

_NC = 2
_NS = 16
_NW = _NC * _NS
_L = 16

_BATCH = 16384
_FEAT = 64
_CLASSES = 100000
_FCHUNK = 4096
_NFC = _BATCH // _FCHUNK
_DPW = _FEAT // _NW


def _sc_partials(features_t, labels, centers_t):
    mesh = plsc.VectorSubcoreMesh(
        core_axis_name="c", subcore_axis_name="s",
        num_cores=_NC, num_subcores=_NS,
    )

    @functools.partial(
        pl.kernel,
        out_type=jax.ShapeDtypeStruct((_NW, _L), jnp.float32),
        mesh=mesh,
        scratch_types=[
            pltpu.VMEM((_CLASSES,), jnp.float32),
            pltpu.VMEM((_BATCH,), jnp.int32),
            pltpu.VMEM((2, _FCHUNK), jnp.float32),
            pltpu.VMEM((_L,), jnp.float32),
            pltpu.SemaphoreType.DMA,
            pltpu.SemaphoreType.DMA,
        ],
        compiler_params=pltpu.CompilerParams(
            use_tc_tiling_on_sc=True, needs_layout_passes=False),
    )
    def k(feat_hbm, lab_hbm, cent_hbm, out_hbm,
          tbl_v, lab_v, fch_v, acc_v, sem_t, sem_f):
        wid = lax.axis_index("s") * _NC + lax.axis_index("c")

        lab_copy = pltpu.async_copy(lab_hbm, lab_v, sem_f)
        lab_copy.wait()

        def feature_dim(r, acc):
            d = wid + r * _NW
            tb = pltpu.async_copy(cent_hbm.at[d], tbl_v, sem_t)
            f0 = pltpu.async_copy(
                feat_hbm.at[d, pl.ds(0, _FCHUNK)], fch_v.at[0], sem_f)
            tb.wait()

            def chunk_body(j, a):
                slot = lax.rem(j, 2)
                nxt = lax.rem(j + 1, 2)
                a = lax.cond(
                    j + 1 < _NFC,
                    lambda x: (pltpu.async_copy(
                        feat_hbm.at[d, pl.ds((j + 1) * _FCHUNK, _FCHUNK)],
                        fch_v.at[nxt], sem_f), x)[1],
                    lambda x: x, a)
                pltpu.make_async_copy(
                    feat_hbm.at[d, pl.ds(0, _FCHUNK)],
                    fch_v.at[slot], sem_f).wait()

                def vec_body(i, a2):
                    idx = lab_v[pl.ds(j * _FCHUNK + i * _L, _L)]
                    cv = plsc.load_gather(tbl_v, [idx])
                    fv = fch_v[slot, pl.ds(i * _L, _L)]
                    dlt = fv - cv
                    return a2 + dlt * dlt

                return lax.fori_loop(0, _FCHUNK // _L, vec_body, a)

            return lax.fori_loop(0, _NFC, chunk_body, acc)

        acc = lax.fori_loop(
            0, _DPW, feature_dim, jnp.zeros((_L,), jnp.float32))
        acc_v[...] = acc
        pltpu.sync_copy(acc_v, out_hbm.at[wid])

    return k(features_t, labels, centers_t)


def _reduce_body(p_ref, o_ref):
    o_ref[0, 0] = jnp.sum(p_ref[...]) * (0.5 / _BATCH)


def _final_reduce(partials):
    out = pl.pallas_call(
        _reduce_body,
        out_shape=jax.ShapeDtypeStruct((1, 1), jnp.float32),
        out_specs=pl.BlockSpec(memory_space=pltpu.SMEM),
    )(partials)
    return out[0, 0]


def kernel(features, labels, centers):
    labels = labels.astype(jnp.int32)
    partials = _sc_partials(features.T, labels, centers.T)
    return _final_reduce(partials)


# speedup vs baseline: 2.3407x; 1.1277x over previous
import functools

import jax
import jax.numpy as jnp
from jax import lax
from jax.experimental import pallas as pl
from jax.experimental.pallas import tpu as pltpu
from jax.experimental.pallas import tpu_sc as plsc

_NC = 2
_NS = 16
_NW = _NC * _NS
_L = 16

_BATCH = 16384
_FEAT = 64
_CLASSES = 100000
_FCHUNK = 4096
_NFC = _BATCH // _FCHUNK
_DPW = _FEAT // _NW
_UNROLL = 8
_NACC = 4


def _sc_partials(features_t, labels, centers_t):
    mesh = plsc.VectorSubcoreMesh(
        core_axis_name="c", subcore_axis_name="s",
        num_cores=_NC, num_subcores=_NS,
    )

    @functools.partial(
        pl.kernel,
        out_type=jax.ShapeDtypeStruct((_NW, _L), jnp.float32),
        mesh=mesh,
        scratch_types=[
            pltpu.VMEM((_CLASSES,), jnp.float32),
            pltpu.VMEM((_BATCH,), jnp.int32),
            pltpu.VMEM((2, _FCHUNK), jnp.float32),
            pltpu.VMEM((_L,), jnp.float32),
            pltpu.SemaphoreType.DMA,
            pltpu.SemaphoreType.DMA,
        ],
        compiler_params=pltpu.CompilerParams(
            use_tc_tiling_on_sc=True, needs_layout_passes=False),
    )
    def k(feat_hbm, lab_hbm, cent_hbm, out_hbm,
          tbl_v, lab_v, fch_v, acc_v, sem_t, sem_f):
        wid = lax.axis_index("s") * _NC + lax.axis_index("c")

        lab_copy = pltpu.async_copy(lab_hbm, lab_v, sem_f)
        lab_copy.wait()

        def feature_dim(r, acc):
            d = wid + r * _NW
            tb = pltpu.async_copy(cent_hbm.at[d], tbl_v, sem_t)
            f0 = pltpu.async_copy(
                feat_hbm.at[d, pl.ds(0, _FCHUNK)], fch_v.at[0], sem_f)
            tb.wait()

            def chunk_body(j, a):
                slot = lax.rem(j, 2)
                nxt = lax.rem(j + 1, 2)
                a = lax.cond(
                    j + 1 < _NFC,
                    lambda x: (pltpu.async_copy(
                        feat_hbm.at[d, pl.ds((j + 1) * _FCHUNK, _FCHUNK)],
                        fch_v.at[nxt], sem_f), x)[1],
                    lambda x: x, a)
                pltpu.make_async_copy(
                    feat_hbm.at[d, pl.ds(0, _FCHUNK)],
                    fch_v.at[slot], sem_f).wait()

                def vec_body(i, a2):
                    a2 = list(a2)
                    for u in range(_UNROLL):
                        off = (i * _UNROLL + u) * _L
                        idx = lab_v[pl.ds(j * _FCHUNK + off, _L)]
                        cv = plsc.load_gather(tbl_v, [idx])
                        fv = fch_v[slot, pl.ds(off, _L)]
                        dlt = fv - cv
                        a2[u % _NACC] = a2[u % _NACC] + dlt * dlt
                    return tuple(a2)

                return lax.fori_loop(
                    0, _FCHUNK // (_L * _UNROLL), vec_body, a)

            return lax.fori_loop(0, _NFC, chunk_body, acc)

        zero = jnp.zeros((_L,), jnp.float32)
        accs = lax.fori_loop(0, _DPW, feature_dim, (zero,) * _NACC)
        acc_v[...] = accs[0] + accs[1] + accs[2] + accs[3]
        pltpu.sync_copy(acc_v, out_hbm.at[wid])

    return k(features_t, labels, centers_t)


def _reduce_body(p_ref, o_ref):
    o_ref[0, 0] = jnp.sum(p_ref[...]) * (0.5 / _BATCH)


def _final_reduce(partials):
    out = pl.pallas_call(
        _reduce_body,
        out_shape=jax.ShapeDtypeStruct((1, 1), jnp.float32),
        out_specs=pl.BlockSpec(memory_space=pltpu.SMEM),
    )(partials)
    return out[0, 0]


def kernel(features, labels, centers):
    labels = labels.astype(jnp.int32)
    partials = _sc_partials(features.T, labels, centers.T)
    return _final_reduce(partials)


# speedup vs baseline: 2.3511x; 1.0045x over previous
import functools

import jax
import jax.numpy as jnp
from jax import lax
from jax.experimental import pallas as pl
from jax.experimental.pallas import tpu as pltpu
from jax.experimental.pallas import tpu_sc as plsc

_NC = 2
_NS = 16
_NW = _NC * _NS
_L = 16

_BATCH = 16384
_FEAT = 64
_CLASSES = 100000
_FCHUNK = 4096
_NFC = _BATCH // _FCHUNK
_DPW = _FEAT // _NW
_UNROLL = 16
_NACC = 8


def _sc_partials(features_t, labels, centers_t):
    mesh = plsc.VectorSubcoreMesh(
        core_axis_name="c", subcore_axis_name="s",
        num_cores=_NC, num_subcores=_NS,
    )

    @functools.partial(
        pl.kernel,
        out_type=jax.ShapeDtypeStruct((_NW, _L), jnp.float32),
        mesh=mesh,
        scratch_types=[
            pltpu.VMEM((_CLASSES,), jnp.float32),
            pltpu.VMEM((_BATCH,), jnp.int32),
            pltpu.VMEM((2, _FCHUNK), jnp.float32),
            pltpu.VMEM((_L,), jnp.float32),
            pltpu.SemaphoreType.DMA,
            pltpu.SemaphoreType.DMA,
        ],
        compiler_params=pltpu.CompilerParams(
            use_tc_tiling_on_sc=True, needs_layout_passes=False),
    )
    def k(feat_hbm, lab_hbm, cent_hbm, out_hbm,
          tbl_v, lab_v, fch_v, acc_v, sem_t, sem_f):
        wid = lax.axis_index("s") * _NC + lax.axis_index("c")

        lab_copy = pltpu.async_copy(lab_hbm, lab_v, sem_f)
        lab_copy.wait()

        def feature_dim(r, acc):
            d = wid + r * _NW
            tb = pltpu.async_copy(cent_hbm.at[d], tbl_v, sem_t)
            f0 = pltpu.async_copy(
                feat_hbm.at[d, pl.ds(0, _FCHUNK)], fch_v.at[0], sem_f)
            tb.wait()

            def chunk_body(j, a):
                slot = lax.rem(j, 2)
                nxt = lax.rem(j + 1, 2)
                a = lax.cond(
                    j + 1 < _NFC,
                    lambda x: (pltpu.async_copy(
                        feat_hbm.at[d, pl.ds((j + 1) * _FCHUNK, _FCHUNK)],
                        fch_v.at[nxt], sem_f), x)[1],
                    lambda x: x, a)
                pltpu.make_async_copy(
                    feat_hbm.at[d, pl.ds(0, _FCHUNK)],
                    fch_v.at[slot], sem_f).wait()

                def vec_body(i, a2):
                    a2 = list(a2)
                    for u in range(_UNROLL):
                        off = (i * _UNROLL + u) * _L
                        idx = lab_v[pl.ds(j * _FCHUNK + off, _L)]
                        cv = plsc.load_gather(tbl_v, [idx])
                        fv = fch_v[slot, pl.ds(off, _L)]
                        dlt = fv - cv
                        a2[u % _NACC] = a2[u % _NACC] + dlt * dlt
                    return tuple(a2)

                return lax.fori_loop(
                    0, _FCHUNK // (_L * _UNROLL), vec_body, a)

            return lax.fori_loop(0, _NFC, chunk_body, acc)

        zero = jnp.zeros((_L,), jnp.float32)
        accs = lax.fori_loop(0, _DPW, feature_dim, (zero,) * _NACC)
        total = accs[0]
        for t in accs[1:]:
            total = total + t
        acc_v[...] = total
        pltpu.sync_copy(acc_v, out_hbm.at[wid])

    return k(features_t, labels, centers_t)


def _reduce_body(p_ref, o_ref):
    o_ref[0, 0] = jnp.sum(p_ref[...]) * (0.5 / _BATCH)


def _final_reduce(partials):
    out = pl.pallas_call(
        _reduce_body,
        out_shape=jax.ShapeDtypeStruct((1, 1), jnp.float32),
        out_specs=pl.BlockSpec(memory_space=pltpu.SMEM),
    )(partials)
    return out[0, 0]


def kernel(features, labels, centers):
    labels = labels.astype(jnp.int32)
    partials = _sc_partials(features.T, labels, centers.T)
    return _final_reduce(partials)
